# baseline (speedup 1.0000x reference)
import functools

import jax
import jax.numpy as jnp
from jax import lax
from jax.experimental import pallas as pl
from jax.experimental.pallas import tpu as pltpu
from jax.experimental.pallas import tpu_sc as plsc

D_MODEL = 64
VOCAB = 66
NJ = 200
LANES = 16
CHUNK_OFFS = tuple(range(0, NJ - LANES + 1, LANES)) + (NJ - LANES,)
NBUF = 3
VOCAB_OFFS = tuple(range(0, VOCAB + LANES - 1, LANES))


@functools.lru_cache(maxsize=None)
def _make_sc_k(nb: int, ni: int):
    info = plsc.get_sparse_core_info()
    nc, ns = info.num_cores, info.num_subcores
    nw = nc * ns
    n_blocks = nb * ni
    assert n_blocks % nw == 0
    blocks_per_w = n_blocks // nw
    mesh = plsc.VectorSubcoreMesh(core_axis_name="c", subcore_axis_name="s")

    @functools.partial(
        pl.kernel,
        mesh=mesh,
        compiler_params=pltpu.CompilerParams(needs_layout_passes=False),
        out_type=jax.ShapeDtypeStruct((nb, ni, D_MODEL, NJ), jnp.float32),
        scratch_types=[
            pltpu.VMEM((VOCAB, 2 * D_MODEL), jnp.float32),
            pltpu.VMEM((D_MODEL * VOCAB * LANES,), jnp.float32),
            pltpu.VMEM((NBUF, NJ), jnp.int32),
            pltpu.VMEM((NBUF * D_MODEL, NJ), jnp.float32),
            pltpu.SemaphoreType.DMA((NBUF,)),
            pltpu.SemaphoreType.DMA((NBUF,)),
        ],
    )
    def sc_k_kernel(idx_hbm, tab_hbm, outk_hbm,
                    tab2, tab_t, idx_v, kblk, semi, semk):
        wid = lax.axis_index("s") * nc + lax.axis_index("c")
        base = wid * blocks_per_w

        def bi(blk):
            return blk // ni, blk % ni

        def start_idx(blk, u):
            b, i = bi(blk)
            pltpu.async_copy(idx_hbm.at[b, i], idx_v.at[u], semi.at[u])

        pltpu.sync_copy(tab_hbm, tab2)
        start_idx(base, 0)
        start_idx(base + 1, 1)
        lane_iota = lax.iota(jnp.int32, LANES)
        cvecs = tuple(lane_iota + co for co in VOCAB_OFFS)
        masks = tuple(cv < VOCAB for cv in cvecs)
        ovecs = tuple(lane_iota * LANES + ((lane_iota + r) % LANES)
                      for r in range(LANES))

        def t_body(d, carry):
            for ci, co in enumerate(VOCAB_OFFS):
                vals = plsc.load_gather(
                    tab2, [cvecs[ci], jnp.full((LANES,), d, jnp.int32)],
                    mask=masks[ci])
                ebase = (d * VOCAB + co) * LANES
                for r in range(LANES):
                    plsc.store_scatter(
                        tab_t, [ovecs[r] + ebase], vals, mask=masks[ci])
            return carry

        lax.fori_loop(0, D_MODEL, t_body, 0, unroll=1)

        def block_body(t, carry):
            u = t % NBUF
            blk = base + t
            b, i = bi(blk)
            kb = kblk.at[pl.ds(u * D_MODEL, D_MODEL)]

            @pl.when(t >= NBUF)
            def _():
                pltpu.make_async_copy(kb, outk_hbm.at[b, i], semk.at[u]).wait()

            pltpu.make_async_copy(idx_hbm.at[b, i], idx_v.at[u],
                                  semi.at[u]).wait()

            addrs0 = tuple(
                idx_v[u, pl.ds(co, LANES)] * LANES + lane_iota
                for co in CHUNK_OFFS)
            n_g = len(addrs0)
            depth = 4

            def d_body(d, addrs):
                row = u * D_MODEL + d
                vals = [None] * n_g
                for g in range(n_g):
                    vals[g] = plsc.load_gather(tab_t, [addrs[g]])
                    if g >= depth:
                        kblk[row, pl.ds(CHUNK_OFFS[g - depth], LANES)] = (
                            vals[g - depth])
                for g in range(n_g - depth, n_g):
                    kblk[row, pl.ds(CHUNK_OFFS[g], LANES)] = vals[g]
                return tuple(a + VOCAB * LANES for a in addrs)

            lax.fori_loop(0, D_MODEL, d_body, addrs0, unroll=2)

            @pl.when(t + 2 < blocks_per_w)
            def _():
                start_idx(blk + 2, (u + 2) % NBUF)

            pltpu.async_copy(kb, outk_hbm.at[b, i], semk.at[u])
            return carry

        lax.fori_loop(0, blocks_per_w, block_body, 0)

        for t in range(blocks_per_w - NBUF, blocks_per_w):
            u = t % NBUF
            b, i = bi(base + t)
            kb = kblk.at[pl.ds(u * D_MODEL, D_MODEL)]
            pltpu.make_async_copy(kb, outk_hbm.at[b, i], semk.at[u]).wait()

    return sc_k_kernel


def _tc_v_body(idx_ref, tabt_ref, out_ref):
    tab_v = tabt_ref[...]
    iota_c = lax.broadcasted_iota(jnp.int32, (VOCAB, NJ), 0)
    for r in range(idx_ref.shape[1]):
        idx = idx_ref[0, r, :]
        onehot = jnp.where(idx[None, :] == iota_c, 1.0, 0.0)
        out_ref[0, r] = lax.dot_general(
            tab_v, onehot, (((1,), (0,)), ((), ())),
            preferred_element_type=jnp.float32)


@functools.lru_cache(maxsize=None)
def _make_tc_v(nb: int, ni: int):
    rows = 40
    grid = (nb, ni // rows)
    return pl.pallas_call(
        _tc_v_body,
        grid=grid,
        in_specs=[
            pl.BlockSpec((1, rows, NJ), lambda b, g: (b, g, 0)),
            pl.BlockSpec((D_MODEL, VOCAB), lambda b, g: (0, 0)),
        ],
        out_specs=pl.BlockSpec((1, rows, D_MODEL, NJ),
                               lambda b, g: (b, g, 0, 0)),
        out_shape=jax.ShapeDtypeStruct((nb, ni, D_MODEL, NJ), jnp.float32),
    )


def kernel(inputs, relation_type, parent_table, brother_table):
    table = parent_table if relation_type == "parent" else brother_table
    nb, ni = inputs.shape[0], inputs.shape[1]
    idx = inputs.astype(jnp.int32)
    outk = _make_sc_k(nb, ni)(idx, table)
    outv = _make_tc_v(nb, ni)(idx, table[:, D_MODEL:].T)
    return (outk.transpose(0, 1, 3, 2), outv.transpose(0, 1, 3, 2))


# speedup vs baseline: 1.0388x; 1.0388x over previous
import functools

import jax
import jax.numpy as jnp
from jax import lax
from jax.experimental import pallas as pl
from jax.experimental.pallas import tpu as pltpu
from jax.experimental.pallas import tpu_sc as plsc

D_MODEL = 64
VOCAB = 66
NJ = 200
LANES = 16
CHUNK_OFFS = tuple(range(0, NJ - LANES + 1, LANES)) + (NJ - LANES,)
NBUF = 4
VOCAB_OFFS = tuple(range(0, VOCAB - LANES + 1, LANES)) + (VOCAB - LANES,)


@functools.lru_cache(maxsize=None)
def _make_sc_k(nb: int, ni: int):
    info = plsc.get_sparse_core_info()
    nc, ns = info.num_cores, info.num_subcores
    nw = nc * ns
    n_blocks = nb * ni
    assert n_blocks % nw == 0
    blocks_per_w = n_blocks // nw
    mesh = plsc.VectorSubcoreMesh(core_axis_name="c", subcore_axis_name="s")

    @functools.partial(
        pl.kernel,
        mesh=mesh,
        compiler_params=pltpu.CompilerParams(needs_layout_passes=False),
        out_type=jax.ShapeDtypeStruct((nb, ni, D_MODEL, NJ), jnp.float32),
        scratch_types=[
            pltpu.VMEM((VOCAB, 2 * D_MODEL), jnp.float32),
            pltpu.VMEM((D_MODEL * VOCAB,), jnp.float32),
            pltpu.VMEM((NBUF, NJ), jnp.int32),
            pltpu.VMEM((NBUF * D_MODEL, NJ), jnp.float32),
            pltpu.SemaphoreType.DMA((NBUF,)),
            pltpu.SemaphoreType.DMA((NBUF,)),
        ],
    )
    def sc_k_kernel(idx_hbm, tab_hbm, outk_hbm,
                    tab2, tab_t, idx_v, kblk, semi, semk):
        wid = lax.axis_index("s") * nc + lax.axis_index("c")
        base = wid * blocks_per_w

        def bi(blk):
            return blk // ni, blk % ni

        def start_idx(blk, u):
            b, i = bi(blk)
            pltpu.async_copy(idx_hbm.at[b, i], idx_v.at[u], semi.at[u])

        pltpu.sync_copy(tab_hbm, tab2)
        start_idx(base, 0)
        start_idx(base + 1, 1)
        lane_iota = lax.iota(jnp.int32, LANES)
        cvecs = tuple(lane_iota + co for co in VOCAB_OFFS)

        def t_body(d, carry):
            for ci, co in enumerate(VOCAB_OFFS):
                vals = plsc.load_gather(
                    tab2, [cvecs[ci], jnp.full((LANES,), d, jnp.int32)])
                tab_t[pl.ds(d * VOCAB + co, LANES)] = vals
            return carry

        lax.fori_loop(0, D_MODEL, t_body, 0, unroll=2)

        def block_body(t, carry):
            u = t % NBUF
            blk = base + t
            b, i = bi(blk)
            kb = kblk.at[pl.ds(u * D_MODEL, D_MODEL)]

            @pl.when(t >= NBUF)
            def _():
                pltpu.make_async_copy(kb, outk_hbm.at[b, i], semk.at[u]).wait()

            pltpu.make_async_copy(idx_hbm.at[b, i], idx_v.at[u],
                                  semi.at[u]).wait()

            addrs0 = tuple(
                idx_v[u, pl.ds(co, LANES)] for co in CHUNK_OFFS)
            n_g = len(addrs0)
            depth = 4

            def d_body(d, addrs):
                row = u * D_MODEL + d
                vals = [None] * n_g
                for g in range(n_g):
                    vals[g] = plsc.load_gather(tab_t, [addrs[g]])
                    if g >= depth:
                        kblk[row, pl.ds(CHUNK_OFFS[g - depth], LANES)] = (
                            vals[g - depth])
                for g in range(n_g - depth, n_g):
                    kblk[row, pl.ds(CHUNK_OFFS[g], LANES)] = vals[g]
                return tuple(a + VOCAB for a in addrs)

            lax.fori_loop(0, D_MODEL, d_body, addrs0, unroll=2)

            @pl.when(t + 2 < blocks_per_w)
            def _():
                start_idx(blk + 2, (u + 2) % NBUF)

            pltpu.async_copy(kb, outk_hbm.at[b, i], semk.at[u])
            return carry

        lax.fori_loop(0, blocks_per_w, block_body, 0)

        for t in range(blocks_per_w - NBUF, blocks_per_w):
            u = t % NBUF
            b, i = bi(base + t)
            kb = kblk.at[pl.ds(u * D_MODEL, D_MODEL)]
            pltpu.make_async_copy(kb, outk_hbm.at[b, i], semk.at[u]).wait()

    return sc_k_kernel


def _tc_v_body(idx_ref, tabt_ref, out_ref):
    tab_v = tabt_ref[...]
    iota_c = lax.broadcasted_iota(jnp.int32, (VOCAB, NJ), 0)
    for r in range(idx_ref.shape[1]):
        idx = idx_ref[0, r, :]
        onehot = jnp.where(idx[None, :] == iota_c, 1.0, 0.0)
        out_ref[0, r] = lax.dot_general(
            tab_v, onehot, (((1,), (0,)), ((), ())),
            preferred_element_type=jnp.float32)


@functools.lru_cache(maxsize=None)
def _make_tc_v(nb: int, ni: int):
    rows = 40
    grid = (nb, ni // rows)
    return pl.pallas_call(
        _tc_v_body,
        grid=grid,
        in_specs=[
            pl.BlockSpec((1, rows, NJ), lambda b, g: (b, g, 0)),
            pl.BlockSpec((D_MODEL, VOCAB), lambda b, g: (0, 0)),
        ],
        out_specs=pl.BlockSpec((1, rows, D_MODEL, NJ),
                               lambda b, g: (b, g, 0, 0)),
        out_shape=jax.ShapeDtypeStruct((nb, ni, D_MODEL, NJ), jnp.float32),
    )


def kernel(inputs, relation_type, parent_table, brother_table):
    table = parent_table if relation_type == "parent" else brother_table
    nb, ni = inputs.shape[0], inputs.shape[1]
    idx = inputs.astype(jnp.int32)
    outk = _make_sc_k(nb, ni)(idx, table)
    outv = _make_tc_v(nb, ni)(idx, table[:, D_MODEL:].T)
    return (outk.transpose(0, 1, 3, 2), outv.transpose(0, 1, 3, 2))
